# Initial kernel scaffold; baseline (speedup 1.0000x reference)
import functools

import jax
import jax.numpy as jnp
from jax import lax
from jax.experimental import pallas as pl
from jax.experimental.pallas import tpu as pltpu
from jax.experimental.pallas import tpu_sc as plsc

B, C, H, W = 32, 32, 224, 224
HW = H * W
L = 16
D = HW // L

_mesh = plsc.VectorSubcoreMesh(core_axis_name="c", subcore_axis_name="s")


def _body(tab_hbm, idx_hbm, out_hbm, idx_v, plane_v, gsem, wsem):
    wid = lax.axis_index("s") * 2 + lax.axis_index("c")
    pltpu.sync_copy(idx_hbm, idx_v)
    widv = jnp.full((L,), wid, dtype=jnp.int32)
    myidx = plsc.load_gather(idx_v, [widv])
    rows = (widv * C + myidx) * L + lax.iota(jnp.int32, (L,), 0)
    pltpu.async_copy(tab_hbm.at[rows], plane_v, gsem).wait()
    copies = [
        pltpu.async_copy(plane_v, out_hbm.at[wid * C + c], wsem)
        for c in range(C)
    ]
    for cp in copies:
        cp.wait()


@functools.partial(
    pl.kernel,
    out_type=jax.ShapeDtypeStruct((B * C, L, D), jnp.float32),
    mesh=_mesh,
    scratch_types=[
        pltpu.VMEM((B,), jnp.int32),
        pltpu.VMEM((L, D), jnp.float32),
        pltpu.SemaphoreType.DMA,
        pltpu.SemaphoreType.DMA,
    ],
)
def _sc_gather_bcast(tab_hbm, idx_hbm, out_hbm, idx_v, plane_v, gsem, wsem):
    _body(tab_hbm, idx_hbm, out_hbm, idx_v, plane_v, gsem, wsem)


def kernel(logits, indices):
    tab = logits.reshape(B * C * L, D)
    idx = indices.astype(jnp.int32)
    out = _sc_gather_bcast(tab, idx)
    return out.reshape(B, C, H, W)


# speedup vs baseline: 6.5305x; 6.5305x over previous
import functools

import jax
import jax.numpy as jnp
from jax import lax
from jax.experimental import pallas as pl
from jax.experimental.pallas import tpu as pltpu
from jax.experimental.pallas import tpu_sc as plsc

B, C, H, W = 32, 32, 224, 224
HW = H * W
L = 8
D = HW // L

_mesh = plsc.VectorSubcoreMesh(core_axis_name="c", subcore_axis_name="s")


@functools.partial(
    pl.kernel,
    out_type=jax.ShapeDtypeStruct((B * C, L, D), jnp.float32),
    mesh=_mesh,
    scratch_types=[
        pltpu.VMEM((L,), jnp.int32),
        pltpu.VMEM((L, D), jnp.float32),
        pltpu.SemaphoreType.DMA,
        pltpu.SemaphoreType.DMA,
    ],
)
def _sc_gather_bcast(tab_hbm, rows_hbm, out_hbm, rows_v, plane_v, gsem, wsem):
    wid = lax.axis_index("s") * 2 + lax.axis_index("c")
    pltpu.sync_copy(rows_hbm.at[wid], rows_v)
    pltpu.async_copy(tab_hbm.at[rows_v], plane_v, gsem).wait()
    copies = [
        pltpu.async_copy(plane_v, out_hbm.at[wid * C + c], wsem)
        for c in range(C)
    ]
    for cp in copies:
        cp.wait()


def kernel(logits, indices):
    tab = logits.reshape(B * C * L, D)
    idx = indices.astype(jnp.int32)
    base = (jnp.arange(B, dtype=jnp.int32) * C + idx) * L
    src_rows = base[:, None] + jnp.arange(L, dtype=jnp.int32)[None, :]
    out = _sc_gather_bcast(tab, src_rows)
    return out.reshape(B, C, H, W)


# speedup vs baseline: 39.1512x; 5.9951x over previous
import functools

import jax
import jax.numpy as jnp
from jax import lax
from jax.experimental import pallas as pl
from jax.experimental.pallas import tpu as pltpu
from jax.experimental.pallas import tpu_sc as plsc

B, C, H, W = 32, 32, 224, 224

_mesh = plsc.VectorSubcoreMesh(core_axis_name="c", subcore_axis_name="s")


@functools.partial(
    pl.kernel,
    out_type=jax.ShapeDtypeStruct((B * C, H, W), jnp.float32),
    mesh=_mesh,
    scratch_types=[
        pltpu.VMEM((16,), jnp.int32),
        pltpu.VMEM((1, H, W), jnp.float32),
        pltpu.SemaphoreType.DMA,
        pltpu.SemaphoreType.DMA,
    ],
)
def _sc_gather_bcast(tab_hbm, rows_hbm, out_hbm, row_v, plane_v, gsem, wsem):
    wid = lax.axis_index("s") * 2 + lax.axis_index("c")
    pltpu.sync_copy(rows_hbm.at[wid], row_v)
    src = row_v[...][0]
    pltpu.async_copy(tab_hbm.at[pl.ds(src, 1)], plane_v, gsem).wait()
    copies = [
        pltpu.async_copy(plane_v, out_hbm.at[pl.ds(wid * C + c, 1)], wsem)
        for c in range(C)
    ]
    for cp in copies:
        cp.wait()


def kernel(logits, indices):
    tab = logits.reshape(B * C, H, W)
    idx = indices.astype(jnp.int32)
    rows = jnp.broadcast_to(
        ((jnp.arange(B, dtype=jnp.int32) * C) + idx)[:, None], (B, 16)
    )
    out = _sc_gather_bcast(tab, rows)
    return out.reshape(B, C, H, W)


# speedup vs baseline: 39.2580x; 1.0027x over previous
import functools

import jax
import jax.numpy as jnp
from jax import lax
from jax.experimental import pallas as pl
from jax.experimental.pallas import tpu as pltpu
from jax.experimental.pallas import tpu_sc as plsc

B, C, H, W = 32, 32, 224, 224

_mesh = plsc.VectorSubcoreMesh(core_axis_name="c", subcore_axis_name="s")


@functools.partial(
    pl.kernel,
    out_type=jax.ShapeDtypeStruct((B * C, H, W), jnp.float32),
    mesh=_mesh,
    scratch_types=[
        pltpu.VMEM((B + 16,), jnp.int32),
        pltpu.VMEM((1, H, W), jnp.float32),
        pltpu.SemaphoreType.DMA,
        pltpu.SemaphoreType.DMA,
    ],
)
def _sc_gather_bcast(tab_hbm, idx_hbm, out_hbm, idx_v, plane_v, gsem, wsem):
    wid = lax.axis_index("s") * 2 + lax.axis_index("c")
    pltpu.sync_copy(idx_hbm, idx_v.at[pl.ds(0, B)])
    src = wid * C + idx_v[pl.ds(wid, 16)][0]
    pltpu.async_copy(tab_hbm.at[pl.ds(src, 1)], plane_v, gsem).wait()
    copies = [
        pltpu.async_copy(plane_v, out_hbm.at[pl.ds(wid * C + c, 1)], wsem)
        for c in range(C)
    ]
    for cp in copies:
        cp.wait()


def kernel(logits, indices):
    tab = logits.reshape(B * C, H, W)
    idx = indices.astype(jnp.int32)
    out = _sc_gather_bcast(tab, idx)
    return out.reshape(B, C, H, W)
